# baseline (speedup 1.0000x reference)
import functools

import jax
import jax.numpy as jnp
from jax import lax
from jax.experimental import pallas as pl
from jax.experimental.pallas import tpu as pltpu
from jax.experimental.pallas import tpu_sc as plsc

N = 10000
D = 128
E = 320000
LANES = 16
NC, NS = 2, 16
HALF = 5120
NPAD = NC * HALF
RPT = HALF // NS
SCAN = E // NS
SSTG = 2000
NSTG = SCAN // SSTG
LIST = 20480
C = 80
NB = 3
PK = 1 << 14
BR = 1024


def _sc_agg(feats, col1, row1, ew1):
    mesh = plsc.VectorSubcoreMesh(core_axis_name="c", subcore_axis_name="s")

    @functools.partial(
        pl.kernel,
        out_type=(
            jax.ShapeDtypeStruct((NPAD, D), jnp.float32),
            jax.ShapeDtypeStruct((NPAD,), jnp.float32),
        ),
        mesh=mesh,
        compiler_params=pltpu.CompilerParams(use_tc_tiling_on_sc=False,
                                             needs_layout_passes=False),
        scratch_types=[
            [pltpu.VMEM((SSTG,), jnp.int32) for _ in range(2)],
            [pltpu.VMEM((SSTG,), jnp.int32) for _ in range(2)],
            [pltpu.VMEM((SSTG,), jnp.float32) for _ in range(2)],
            pltpu.VMEM((LIST,), jnp.int32),
            pltpu.VMEM((LIST,), jnp.float32),
            [pltpu.VMEM((C, D), jnp.float32) for _ in range(NB)],
            [pltpu.VMEM((C,), jnp.int32) for _ in range(NB)],
            [pltpu.VMEM((C,), jnp.int32) for _ in range(NB)],
            [pltpu.VMEM((C,), jnp.float32) for _ in range(NB)],
            pltpu.VMEM((RPT,), jnp.float32),
            pltpu.VMEM_SHARED((HALF, D), jnp.float32),
            pltpu.VMEM_SHARED((HALF,), jnp.float32),
            [[pltpu.SemaphoreType.DMA for _ in range(3)] for _ in range(2)],
            [pltpu.SemaphoreType.DMA for _ in range(NB)],
            [pltpu.SemaphoreType.DMA for _ in range(NB)],
            [pltpu.SemaphoreType.DMA for _ in range(NB)],
        ],
    )
    def body(feats_hbm, col_hbm, row_hbm, ew_hbm, out_hbm, deg_hbm,
             scols, srows, sews, cpk, cew, gbufs, colbs, rowbs, ewbs,
             zdbuf, acc, dacc, semsc, semg, semf, semd):
        cid = lax.axis_index("c")
        sid = lax.axis_index("s")

        lo = cid * HALF
        lo_v = jnp.full((LANES,), lo, jnp.int32)
        hi_v = lo_v + HALF

        zero16 = jnp.zeros((LANES,), jnp.float32)
        zero16i = jnp.zeros((LANES,), jnp.int32)

        def zrow(r, carry):
            for j in range(D // LANES):
                gbufs[0][r, pl.ds(j * LANES, LANES)] = zero16
            return carry

        lax.fori_loop(0, C, zrow, 0)

        def zdeg(r, carry):
            zdbuf[pl.ds(r * LANES, LANES)] = zero16
            return carry

        lax.fori_loop(0, RPT // LANES, zdeg, 0)

        for t in range(RPT // C):
            pltpu.sync_copy(gbufs[0].at[pl.ds(0, C)],
                            acc.at[pl.ds(sid * RPT + t * C, C)])
        pltpu.sync_copy(zdbuf, dacc.at[pl.ds(sid * RPT, RPT)])

        plsc.subcore_barrier()

        def issue_stage(j, p):
            base = sid * SCAN + j * SSTG
            pltpu.async_copy(col_hbm.at[pl.ds(base, SSTG)], scols[p],
                             semsc[p][0])
            pltpu.async_copy(row_hbm.at[pl.ds(base, SSTG)], srows[p],
                             semsc[p][1])
            pltpu.async_copy(ew_hbm.at[pl.ds(base, SSTG)], sews[p],
                             semsc[p][2])

        def wait_stage(p):
            pltpu.make_async_copy(col_hbm.at[pl.ds(0, SSTG)], scols[p],
                                  semsc[p][0]).wait()
            pltpu.make_async_copy(row_hbm.at[pl.ds(0, SSTG)], srows[p],
                                  semsc[p][1]).wait()
            pltpu.make_async_copy(ew_hbm.at[pl.ds(0, SSTG)], sews[p],
                                  semsc[p][2]).wait()

        def process_stage(p, n0):
            def scan_group(g, n):
                col16 = scols[p][pl.ds(g * LANES, LANES)]
                row16 = srows[p][pl.ds(g * LANES, LANES)]
                ew16 = sews[p][pl.ds(g * LANES, LANES)]
                m = (row16 >= lo_v) & (row16 < hi_v)
                mi = lax.select(m, jnp.ones((LANES,), jnp.int32),
                                jnp.zeros((LANES,), jnp.int32))
                pc = plsc.cumsum(mi)
                pos = pc + lax.broadcast(n - 1, (LANES,))
                pk = (row16 - lo_v) * PK + col16
                plsc.store_scatter(cpk, [pos], pk, mask=m)
                plsc.store_scatter(cew, [pos], ew16, mask=m)
                return n + pc[LANES - 1]

            return lax.fori_loop(0, SSTG // LANES, scan_group, n0)

        issue_stage(jnp.int32(0), 0)

        def scan_pair(jj, n):
            j0 = 2 * jj
            wait_stage(0)
            issue_stage(j0 + 1, 1)
            n = process_stage(0, n)
            wait_stage(1)
            issue_stage(jnp.minimum(j0 + 2, NSTG - 1), 0)
            return process_stage(1, n)

        n = lax.fori_loop(0, NSTG // 2, scan_pair, jnp.int32(0))
        wait_stage(0)

        for t in range(NB * C // LANES):
            sl = pl.ds(n + t * LANES, LANES)
            cpk[sl] = zero16i
            cew[sl] = zero16

        nch = (n + C - 1) // C
        ntri = jnp.maximum((nch + NB - 1) // NB, 1)
        last = ntri * NB - 1

        def unpack(k, b):
            def up(g, carry):
                sl = pl.ds(g * LANES, LANES)
                v = cpk[pl.ds(k * C + g * LANES, LANES)]
                rw = v // PK
                rowbs[b][sl] = rw
                colbs[b][sl] = v - rw * PK
                ewbs[b][sl] = cew[pl.ds(k * C + g * LANES, LANES)]
                return carry

            lax.fori_loop(0, C // LANES, up, 0)

        def issue_gather(b):
            pltpu.async_copy(feats_hbm.at[colbs[b]], gbufs[b], semg[b])

        def wait_gather(b):
            pltpu.make_async_copy(feats_hbm.at[colbs[b]], gbufs[b],
                                  semg[b]).wait()

        def issue_scat(b):
            pltpu.async_copy(gbufs[b], acc.at[rowbs[b]], semf[b], add=True)
            pltpu.async_copy(ewbs[b], dacc.at[rowbs[b]], semd[b], add=True)

        def wait_scat(b):
            pltpu.make_async_copy(gbufs[b], acc.at[rowbs[b]], semf[b]).wait()
            pltpu.make_async_copy(ewbs[b], dacc.at[rowbs[b]], semd[b]).wait()

        def scale(b):
            def group_body(g, carry):
                wvec = ewbs[b][pl.ds(g * LANES, LANES)]
                for i in range(LANES):
                    e = g * LANES + i
                    wv = lax.broadcast(wvec[i], (LANES,))
                    for j in range(D // LANES):
                        sl = pl.ds(j * LANES, LANES)
                        gbufs[b][e, sl] = gbufs[b][e, sl] * wv
                return carry

            lax.fori_loop(0, C // LANES, group_body, 0)

        for b in range(NB - 1):
            unpack(jnp.int32(b), b)
            issue_gather(b)

        def tri(i, carry):
            for b in range(NB):
                k = i * NB + b
                wait_gather(b)
                scale(b)
                issue_scat(b)
                bp = (b - 1) % NB
                if b == 0:
                    @pl.when(i > 0)
                    def _():
                        wait_scat(bp)
                else:
                    wait_scat(bp)
                unpack(jnp.minimum(k - 1 + NB, last), bp)
                issue_gather(bp)
            return carry

        lax.fori_loop(0, ntri, tri, 0)

        for b in range(NB - 1):
            wait_gather(b)
        wait_scat(NB - 1)

        plsc.subcore_barrier()

        r0 = sid * RPT
        pltpu.sync_copy(acc.at[pl.ds(r0, RPT)],
                        out_hbm.at[pl.ds(lo + r0, RPT)])
        pltpu.sync_copy(dacc.at[pl.ds(r0, RPT)],
                        deg_hbm.at[pl.ds(lo + r0, RPT)])

    return body(feats, col1, row1, ew1)


def _tc_body(pa_ref, dp_ref, f_ref, w_ref, b_ref, o_ref):
    x = pa_ref[...]
    deg = dp_ref[...]
    h = x / deg
    y = lax.dot_general(h, w_ref[...], (((1,), (1,)), ((), ())),
                        preferred_element_type=jnp.float32)
    o_ref[...] = f_ref[...] + jnp.maximum(y + b_ref[...], 0.0)


def _tc_post(part, degp, feats, W, b2):
    return pl.pallas_call(
        _tc_body,
        grid=(NPAD // BR,),
        in_specs=[
            pl.BlockSpec((BR, D), lambda i: (i, 0)),
            pl.BlockSpec((BR, 1), lambda i: (i, 0)),
            pl.BlockSpec((BR, D), lambda i: (i, 0)),
            pl.BlockSpec((D, D), lambda i: (0, 0)),
            pl.BlockSpec((1, D), lambda i: (0, 0)),
        ],
        out_specs=pl.BlockSpec((BR, D), lambda i: (i, 0)),
        out_shape=jax.ShapeDtypeStruct((N, D), jnp.float32),
    )(part, degp, feats, W, b2)


@jax.jit
def kernel(feats, edge_index, edge_weight, W, b):
    part, degp = _sc_agg(feats, edge_index[1], edge_index[0], edge_weight)
    return _tc_post(part, degp.reshape(NPAD, 1), feats, W, b.reshape(1, D))
